# speedup vs baseline: 1.7710x; 1.7710x over previous
import jax
import jax.numpy as jnp
from jax.experimental import pallas as pl


def _add_rows_kernel(emb_ref, m0_ref, m1_ref, m2_ref, o0_ref, o1_ref, o2_ref):
    o0_ref[...] = m0_ref[...] + emb_ref[0:1, :]
    o1_ref[...] = m1_ref[...] + emb_ref[1:2, :]
    o2_ref[...] = m2_ref[...] + emb_ref[2:3, :]


def kernel(mod0, mod1, mod2, emb):
    B, L, D = mod0.shape
    N = B * L
    R = 512
    x0 = mod0.reshape(N, D)
    x1 = mod1.reshape(N, D)
    x2 = mod2.reshape(N, D)
    row_spec = pl.BlockSpec((R, D), lambda i: (i, 0))
    outs = pl.pallas_call(
        _add_rows_kernel,
        grid=(N // R,),
        in_specs=[
            pl.BlockSpec((emb.shape[0], D), lambda i: (0, 0)),
            row_spec, row_spec, row_spec,
        ],
        out_specs=[row_spec, row_spec, row_spec],
        out_shape=[jax.ShapeDtypeStruct((N, D), jnp.float32)] * 3,
    )(emb, x0, x1, x2)
    return tuple(o.reshape(B, L, D) for o in outs)


# speedup vs baseline: 1.8127x; 1.0236x over previous
import jax
import jax.numpy as jnp
from jax.experimental import pallas as pl


def _add_rows_kernel(emb_ref, m0_ref, m1_ref, m2_ref, o0_ref, o1_ref, o2_ref):
    o0_ref[...] = m0_ref[...] + emb_ref[0:1, :]
    o1_ref[...] = m1_ref[...] + emb_ref[1:2, :]
    o2_ref[...] = m2_ref[...] + emb_ref[2:3, :]


def kernel(mod0, mod1, mod2, emb):
    B, L, D = mod0.shape
    N = B * L
    R = 1024
    x0 = mod0.reshape(N, D)
    x1 = mod1.reshape(N, D)
    x2 = mod2.reshape(N, D)
    row_spec = pl.BlockSpec((R, D), lambda i: (i, 0))
    outs = pl.pallas_call(
        _add_rows_kernel,
        grid=(N // R,),
        in_specs=[
            pl.BlockSpec((emb.shape[0], D), lambda i: (0, 0)),
            row_spec, row_spec, row_spec,
        ],
        out_specs=[row_spec, row_spec, row_spec],
        out_shape=[jax.ShapeDtypeStruct((N, D), jnp.float32)] * 3,
    )(emb, x0, x1, x2)
    return tuple(o.reshape(B, L, D) for o in outs)
